# baseline (speedup 1.0000x reference)
import functools

import jax
import jax.numpy as jnp
from jax.experimental import pallas as pl
from jax.experimental.pallas import tpu as pltpu


def _proposal_kernel(phi_ref, h_ref, a_ref, w_phi_ref, w_h_ref, w_a_ref,
                     b1_ref, w_mean_ref, w_std_ref, b_mean_ref, b_std_ref,
                     mean_ref, std_ref):
    bf16 = jnp.bfloat16
    t = (jnp.dot(phi_ref[...].astype(bf16), w_phi_ref[...],
                 preferred_element_type=jnp.float32)
         + jnp.dot(h_ref[...].astype(bf16), w_h_ref[...],
                   preferred_element_type=jnp.float32)
         + jnp.dot(a_ref[...].astype(bf16), w_a_ref[...],
                   preferred_element_type=jnp.float32)
         + b1_ref[...])
    t = jnp.maximum(t.astype(bf16), 0)

    mean_ref[...] = (jnp.dot(t, w_mean_ref[...],
                             preferred_element_type=jnp.float32)
                     + b_mean_ref[...])
    ys = (jnp.dot(t, w_std_ref[...], preferred_element_type=jnp.float32)
          + b_std_ref[...])
    std_ref[...] = jnp.maximum(ys, 0.0) + jnp.log1p(jnp.exp(-jnp.abs(ys)))


def kernel(phi_x, h, encoded_action, w_phi, w_h, w_a, b1, w_head, b_head):
    B, P, phi_dim = phi_x.shape
    h_dim = h.shape[-1]
    a_dim = encoded_action.shape[-1]
    z_dim = b_head.shape[-1] // 2
    N = B * P

    phi_flat = phi_x.reshape(N, phi_dim)
    h_flat = h.reshape(N, h_dim)
    a_flat = encoded_action.reshape(N, a_dim)

    tile = 4096
    while tile > 8 and N % tile != 0:
        tile //= 2
    n_pad = pl.cdiv(N, tile) * tile
    if n_pad != N:
        pad = n_pad - N
        phi_flat = jnp.pad(phi_flat, ((0, pad), (0, 0)))
        h_flat = jnp.pad(h_flat, ((0, pad), (0, 0)))
        a_flat = jnp.pad(a_flat, ((0, pad), (0, 0)))

    bf16 = jnp.bfloat16
    w_phi_b = w_phi.astype(bf16)
    w_h_b = w_h.astype(bf16)
    w_a_b = w_a.astype(bf16)
    w_mean_b = w_head[:, :z_dim].astype(bf16)
    w_std_b = w_head[:, z_dim:].astype(bf16)
    b_mean = b_head[:, :z_dim]
    b_std = b_head[:, z_dim:]

    grid = (n_pad // tile,)

    mean, std = pl.pallas_call(
        _proposal_kernel,
        out_shape=[
            jax.ShapeDtypeStruct((n_pad, z_dim), phi_x.dtype),
            jax.ShapeDtypeStruct((n_pad, z_dim), phi_x.dtype),
        ],
        grid=grid,
        in_specs=[
            pl.BlockSpec((tile, phi_dim), lambda i: (i, 0)),
            pl.BlockSpec((tile, h_dim), lambda i: (i, 0)),
            pl.BlockSpec((tile, a_dim), lambda i: (i, 0)),
            pl.BlockSpec((phi_dim, h_dim), lambda i: (0, 0)),
            pl.BlockSpec((h_dim, h_dim), lambda i: (0, 0)),
            pl.BlockSpec((a_dim, h_dim), lambda i: (0, 0)),
            pl.BlockSpec((1, h_dim), lambda i: (0, 0)),
            pl.BlockSpec((h_dim, z_dim), lambda i: (0, 0)),
            pl.BlockSpec((h_dim, z_dim), lambda i: (0, 0)),
            pl.BlockSpec((1, z_dim), lambda i: (0, 0)),
            pl.BlockSpec((1, z_dim), lambda i: (0, 0)),
        ],
        out_specs=[
            pl.BlockSpec((tile, z_dim), lambda i: (i, 0)),
            pl.BlockSpec((tile, z_dim), lambda i: (i, 0)),
        ],
        compiler_params=pltpu.CompilerParams(
            dimension_semantics=("parallel",),
            vmem_limit_bytes=128 << 20,
        ),
    )(phi_flat, h_flat, a_flat,
      w_phi_b, w_h_b, w_a_b, b1,
      w_mean_b, w_std_b, b_mean, b_std)

    mean = mean[:N].reshape(B, P, z_dim)
    std = std[:N].reshape(B, P, z_dim)
    return mean, std


# speedup vs baseline: 1.0188x; 1.0188x over previous
import functools

import jax
import jax.numpy as jnp
from jax.experimental import pallas as pl
from jax.experimental.pallas import tpu as pltpu


def _proposal_kernel(phi_ref, h_ref, a_ref, w_phi_ref, w_h_ref, w_a_ref,
                     b1_ref, w_mean_ref, w_std_ref, b_mean_ref, b_std_ref,
                     mean_ref, std_ref):
    bf16 = jnp.bfloat16
    t = (jnp.dot(phi_ref[...].astype(bf16), w_phi_ref[...],
                 preferred_element_type=jnp.float32)
         + jnp.dot(h_ref[...].astype(bf16), w_h_ref[...],
                   preferred_element_type=jnp.float32)
         + jnp.dot(a_ref[...].astype(bf16), w_a_ref[...],
                   preferred_element_type=jnp.float32)
         + b1_ref[...])
    t = jnp.maximum(t, 0.0).astype(bf16)

    mean_ref[...] = (jnp.dot(t, w_mean_ref[...],
                             preferred_element_type=jnp.float32)
                     + b_mean_ref[...])
    ys = (jnp.dot(t, w_std_ref[...], preferred_element_type=jnp.float32)
          + b_std_ref[...])
    std_ref[...] = jnp.maximum(ys, 0.0) + jnp.log1p(jnp.exp(-jnp.abs(ys)))


def kernel(phi_x, h, encoded_action, w_phi, w_h, w_a, b1, w_head, b_head):
    B, P, phi_dim = phi_x.shape
    h_dim = h.shape[-1]
    a_dim = encoded_action.shape[-1]
    z_dim = b_head.shape[-1] // 2
    N = B * P

    phi_flat = phi_x.reshape(N, phi_dim)
    h_flat = h.reshape(N, h_dim)
    a_flat = encoded_action.reshape(N, a_dim)

    tile = 4096
    while tile > 8 and N % tile != 0:
        tile //= 2
    n_pad = pl.cdiv(N, tile) * tile
    if n_pad != N:
        pad = n_pad - N
        phi_flat = jnp.pad(phi_flat, ((0, pad), (0, 0)))
        h_flat = jnp.pad(h_flat, ((0, pad), (0, 0)))
        a_flat = jnp.pad(a_flat, ((0, pad), (0, 0)))

    bf16 = jnp.bfloat16
    w_phi_b = w_phi.astype(bf16)
    w_h_b = w_h.astype(bf16)
    w_a_b = w_a.astype(bf16)
    w_mean_b = w_head[:, :z_dim].astype(bf16)
    w_std_b = w_head[:, z_dim:].astype(bf16)
    b_mean = b_head[:, :z_dim]
    b_std = b_head[:, z_dim:]

    grid = (n_pad // tile,)

    mean, std = pl.pallas_call(
        _proposal_kernel,
        out_shape=[
            jax.ShapeDtypeStruct((n_pad, z_dim), phi_x.dtype),
            jax.ShapeDtypeStruct((n_pad, z_dim), phi_x.dtype),
        ],
        grid=grid,
        in_specs=[
            pl.BlockSpec((tile, phi_dim), lambda i: (i, 0)),
            pl.BlockSpec((tile, h_dim), lambda i: (i, 0)),
            pl.BlockSpec((tile, a_dim), lambda i: (i, 0)),
            pl.BlockSpec((phi_dim, h_dim), lambda i: (0, 0)),
            pl.BlockSpec((h_dim, h_dim), lambda i: (0, 0)),
            pl.BlockSpec((a_dim, h_dim), lambda i: (0, 0)),
            pl.BlockSpec((1, h_dim), lambda i: (0, 0)),
            pl.BlockSpec((h_dim, z_dim), lambda i: (0, 0)),
            pl.BlockSpec((h_dim, z_dim), lambda i: (0, 0)),
            pl.BlockSpec((1, z_dim), lambda i: (0, 0)),
            pl.BlockSpec((1, z_dim), lambda i: (0, 0)),
        ],
        out_specs=[
            pl.BlockSpec((tile, z_dim), lambda i: (i, 0)),
            pl.BlockSpec((tile, z_dim), lambda i: (i, 0)),
        ],
        compiler_params=pltpu.CompilerParams(
            dimension_semantics=("parallel",),
            vmem_limit_bytes=128 << 20,
        ),
    )(phi_flat, h_flat, a_flat,
      w_phi_b, w_h_b, w_a_b, b1,
      w_mean_b, w_std_b, b_mean, b_std)

    mean = mean[:N].reshape(B, P, z_dim)
    std = std[:N].reshape(B, P, z_dim)
    return mean, std


# speedup vs baseline: 1.0292x; 1.0102x over previous
import functools

import jax
import jax.numpy as jnp
from jax.experimental import pallas as pl
from jax.experimental.pallas import tpu as pltpu


def _proposal_kernel(phi_ref, h_ref, a_ref, w1_ref,
                     b1_ref, w_mean_ref, w_std_ref, b_mean_ref, b_std_ref,
                     mean_ref, std_ref):
    bf16 = jnp.bfloat16
    x = jnp.concatenate([phi_ref[...].astype(bf16),
                         h_ref[...].astype(bf16),
                         a_ref[...].astype(bf16)], axis=1)
    t = jnp.dot(x, w1_ref[...], preferred_element_type=jnp.float32) + b1_ref[...]
    t = jnp.maximum(t, 0.0).astype(bf16)

    mean_ref[...] = (jnp.dot(t, w_mean_ref[...],
                             preferred_element_type=jnp.float32)
                     + b_mean_ref[...])
    ys = (jnp.dot(t, w_std_ref[...], preferred_element_type=jnp.float32)
          + b_std_ref[...])
    std_ref[...] = jnp.maximum(ys, 0.0) + jnp.log1p(jnp.exp(-jnp.abs(ys)))


def kernel(phi_x, h, encoded_action, w_phi, w_h, w_a, b1, w_head, b_head):
    B, P, phi_dim = phi_x.shape
    h_dim = h.shape[-1]
    a_dim = encoded_action.shape[-1]
    z_dim = b_head.shape[-1] // 2
    N = B * P

    phi_flat = phi_x.reshape(N, phi_dim)
    h_flat = h.reshape(N, h_dim)
    a_flat = encoded_action.reshape(N, a_dim)

    tile = 4096
    while tile > 8 and N % tile != 0:
        tile //= 2
    n_pad = pl.cdiv(N, tile) * tile
    if n_pad != N:
        pad = n_pad - N
        phi_flat = jnp.pad(phi_flat, ((0, pad), (0, 0)))
        h_flat = jnp.pad(h_flat, ((0, pad), (0, 0)))
        a_flat = jnp.pad(a_flat, ((0, pad), (0, 0)))

    bf16 = jnp.bfloat16
    w1_b = jnp.concatenate([w_phi, w_h, w_a], axis=0).astype(bf16)
    w_mean_b = w_head[:, :z_dim].astype(bf16)
    w_std_b = w_head[:, z_dim:].astype(bf16)
    b_mean = b_head[:, :z_dim]
    b_std = b_head[:, z_dim:]

    grid = (n_pad // tile,)

    mean, std = pl.pallas_call(
        _proposal_kernel,
        out_shape=[
            jax.ShapeDtypeStruct((n_pad, z_dim), phi_x.dtype),
            jax.ShapeDtypeStruct((n_pad, z_dim), phi_x.dtype),
        ],
        grid=grid,
        in_specs=[
            pl.BlockSpec((tile, phi_dim), lambda i: (i, 0)),
            pl.BlockSpec((tile, h_dim), lambda i: (i, 0)),
            pl.BlockSpec((tile, a_dim), lambda i: (i, 0)),
            pl.BlockSpec((phi_dim + h_dim + a_dim, h_dim), lambda i: (0, 0)),
            pl.BlockSpec((1, h_dim), lambda i: (0, 0)),
            pl.BlockSpec((h_dim, z_dim), lambda i: (0, 0)),
            pl.BlockSpec((h_dim, z_dim), lambda i: (0, 0)),
            pl.BlockSpec((1, z_dim), lambda i: (0, 0)),
            pl.BlockSpec((1, z_dim), lambda i: (0, 0)),
        ],
        out_specs=[
            pl.BlockSpec((tile, z_dim), lambda i: (i, 0)),
            pl.BlockSpec((tile, z_dim), lambda i: (i, 0)),
        ],
        compiler_params=pltpu.CompilerParams(
            dimension_semantics=("parallel",),
            vmem_limit_bytes=128 << 20,
        ),
    )(phi_flat, h_flat, a_flat,
      w1_b, b1,
      w_mean_b, w_std_b, b_mean, b_std)

    mean = mean[:N].reshape(B, P, z_dim)
    std = std[:N].reshape(B, P, z_dim)
    return mean, std
